# baseline (speedup 1.0000x reference)
import functools

import jax
import jax.numpy as jnp
from jax import lax
from jax.experimental import pallas as pl
from jax.experimental.pallas import tpu as pltpu
from jax.experimental.pallas import tpu_sc as plsc

DIM = 64
IDXV = 128
SUB = 4
CHUNK = IDXV * SUB


@functools.partial(jax.jit, static_argnums=(2, 3, 4))
def _sc_gather(idx, weights, nw, nc, chunks):
    mesh = plsc.VectorSubcoreMesh(core_axis_name="c", subcore_axis_name="s")
    rpw = chunks * CHUNK
    n = nw * rpw

    @functools.partial(
        pl.kernel,
        mesh=mesh,
        out_type=jax.ShapeDtypeStruct((n, DIM), jnp.float32),
        scratch_types=[
            pltpu.VMEM((chunks * SUB, IDXV), jnp.int32),
            pltpu.VMEM((2, CHUNK, DIM), jnp.float32),
            pltpu.SemaphoreType.DMA,
            pltpu.SemaphoreType.DMA,
            pltpu.SemaphoreType.DMA,
            pltpu.SemaphoreType.DMA,
        ],
        compiler_params=pltpu.CompilerParams(use_tc_tiling_on_sc=False),
    )
    def body(idx_hbm, table_hbm, out_hbm, idx_v, rows_v, g0, g1, s0, s1):
        wid = lax.axis_index("s") * nc + lax.axis_index("c")
        pltpu.sync_copy(idx_hbm.at[wid], idx_v)
        row_base = wid * rpw
        gsem = (g0, g1)
        ssem = (s0, s1)

        def gather(jj, b):
            return [
                pltpu.make_async_copy(
                    table_hbm.at[idx_v.at[jj * SUB + s]],
                    rows_v.at[b].at[pl.ds(s * IDXV, IDXV)],
                    gsem[b],
                )
                for s in range(SUB)
            ]

        def store(jj, b):
            return pltpu.make_async_copy(
                rows_v.at[b],
                out_hbm.at[pl.ds(row_base + jj * CHUNK, CHUNK)],
                ssem[b],
            )

        for d in gather(0, 0):
            d.start()

        def outer(g, carry):
            for b in range(2):
                jj = 2 * g + b
                nb = b ^ 1
                @pl.when(jj >= 1)
                def _():
                    store(jj - 1, nb).wait()

                @pl.when(jj + 1 < chunks)
                def _():
                    for d in gather(jj + 1, nb):
                        d.start()

                for d in gather(jj, b):
                    d.wait()
                store(jj, b).start()
            return carry

        lax.fori_loop(0, chunks // 2, outer, 0)
        store(chunks - 1, (chunks - 1) % 2).wait()

    return body(idx, weights)


def kernel(x, weights):
    b, f = x.shape
    n = b * f
    info = plsc.get_sparse_core_info()
    nw = info.num_cores * info.num_subcores
    chunks = n // (nw * CHUNK)
    idx = x.reshape(nw, chunks * SUB, IDXV).astype(jnp.int32)
    out = _sc_gather(idx, weights, nw, info.num_cores, chunks)
    return out.reshape(b, f, DIM)
